# baseline (speedup 1.0000x reference)
import functools

import jax
import jax.numpy as jnp
from jax import lax
from jax.experimental import pallas as pl
from jax.experimental.pallas import tpu as pltpu
from jax.experimental.pallas import tpu_sc as plsc

HIST = 50
DM = 64
PADW = 128
GROUP = 128
NBUF = 5
K = 3
NW = 32
TBLK = 8192
BJC = 1024


@functools.partial(jax.jit, static_argnums=(2,))
def _sc_gather(w_pad, idx_grouped, ngroups):
    b_total = NW * ngroups * GROUP
    mesh = plsc.VectorSubcoreMesh(core_axis_name="c", subcore_axis_name="s")
    nc = plsc.get_sparse_core_info().num_cores

    @functools.partial(
        pl.kernel,
        mesh=mesh,
        out_type=jax.ShapeDtypeStruct((b_total, PADW), jnp.float32),
        scratch_types=[
            pltpu.VMEM((ngroups, GROUP), jnp.int32),
            pltpu.VMEM((NBUF, GROUP, PADW), jnp.float32),
            pltpu.SemaphoreType.DMA((NBUF,)),
            pltpu.SemaphoreType.DMA((NBUF,)),
        ],
        compiler_params=pltpu.CompilerParams(use_tc_tiling_on_sc=False),
    )
    def k(w_hbm, idx_hbm, out_hbm, idx_v, rows_v, gsem, ssem):
        wid = lax.axis_index("s") * nc + lax.axis_index("c")
        base = wid * (ngroups * GROUP)
        pltpu.sync_copy(idx_hbm.at[wid], idx_v)

        def gather_start(g, b):
            pltpu.async_copy(w_hbm.at[idx_v.at[g]], rows_v.at[b], gsem.at[b])

        def gather_wait(g, b):
            pltpu.make_async_copy(w_hbm.at[idx_v.at[g]], rows_v.at[b],
                                  gsem.at[b]).wait()

        def scat_start(g, b):
            pltpu.async_copy(rows_v.at[b],
                             out_hbm.at[pl.ds(base + g * GROUP, GROUP)],
                             ssem.at[b])

        def scat_wait(g, b):
            pltpu.make_async_copy(rows_v.at[b],
                                  out_hbm.at[pl.ds(base + g * GROUP, GROUP)],
                                  ssem.at[b]).wait()

        for b in range(K):
            gather_start(b, b)

        def outer(t, _):
            for j in range(NBUF):
                g = t * NBUF + j
                gather_wait(g, j)
                scat_start(g, j)
                gn = g + K
                bn = (j + K) % NBUF

                @pl.when(gn < ngroups)
                def _():
                    @pl.when(gn >= NBUF)
                    def _():
                        scat_wait(gn - NBUF, bn)

                    gather_start(gn, bn)

            return 0

        lax.fori_loop(0, ngroups // NBUF, outer, 0)
        for b in range(NBUF):
            scat_wait(ngroups - NBUF + b, b)

    return k(w_pad, idx_grouped)


def _tc_pad_table_body(wt_ref, o_ref):
    blk = wt_ref[...]
    o_ref[...] = jnp.concatenate(
        [blk.T, jnp.zeros((blk.shape[1], PADW - DM), jnp.float32)], axis=1)


@jax.jit
def _tc_pad_table(wt):
    v = wt.shape[1]
    grid = (v + TBLK - 1) // TBLK
    return pl.pallas_call(
        _tc_pad_table_body,
        grid=(grid,),
        in_specs=[pl.BlockSpec((DM, TBLK), lambda c: (0, c))],
        out_specs=pl.BlockSpec((TBLK, PADW), lambda c: (c, 0)),
        out_shape=jax.ShapeDtypeStruct((v, PADW), jnp.float32),
    )(wt)


def _tc_transpose_body(x_ref, o_ref):
    blk = x_ref[0]
    eye = jnp.eye(DM, dtype=jnp.float32)
    xt = lax.dot_general(eye, blk[:, :DM], (((1,), (1,)), ((), ())),
                         preferred_element_type=jnp.float32)
    o_ref[0] = xt.reshape(8, 8, 8, GROUP).transpose(0, 2, 1, 3)


@jax.jit
def _tc_transpose(x):
    bsz = x.shape[1]
    nbj = bsz // GROUP
    return pl.pallas_call(
        _tc_transpose_body,
        grid=(HIST, bsz // BJC),
        in_specs=[pl.BlockSpec((1, BJC, PADW), lambda h, c: (h, c, 0))],
        out_specs=pl.BlockSpec((1, 8, BJC // GROUP, 8, GROUP),
                               lambda h, c: (h, 0, c, 0, 0)),
        out_shape=jax.ShapeDtypeStruct((HIST, 8, nbj, 8, GROUP), jnp.float32),
    )(x)


def kernel(token_ids, weight):
    bsz, h = token_ids.shape
    total = bsz * h
    ngroups = total // (NW * GROUP)
    idx = token_ids.astype(jnp.int32).T.reshape(NW, ngroups, GROUP)
    w_pad = _tc_pad_table(weight.T)
    rows = _sc_gather(w_pad, idx, ngroups)
    out5 = _tc_transpose(rows.reshape(h, bsz, PADW))
    return out5.transpose(2, 4, 0, 1, 3).reshape(bsz, h, DM)


# speedup vs baseline: 1.5122x; 1.5122x over previous
import functools

import jax
import jax.numpy as jnp
from jax import lax
from jax.experimental import pallas as pl
from jax.experimental.pallas import tpu as pltpu
from jax.experimental.pallas import tpu_sc as plsc

HIST = 50
DM = 64
PADW = 128
GROUP = 128
NBUF = 5
K = 3
NW = 32
TBLK = 8192
BJC = 1024


@functools.partial(jax.jit, static_argnums=(2,))
def _sc_gather(w_pad, idx_grouped, ngroups):
    b_total = NW * ngroups * GROUP
    mesh = plsc.VectorSubcoreMesh(core_axis_name="c", subcore_axis_name="s")
    nc = plsc.get_sparse_core_info().num_cores

    @functools.partial(
        pl.kernel,
        mesh=mesh,
        out_type=jax.ShapeDtypeStruct((b_total, PADW), jnp.float32),
        scratch_types=[
            pltpu.VMEM((ngroups, GROUP), jnp.int32),
            pltpu.VMEM((NBUF, GROUP, PADW), jnp.float32),
            pltpu.SemaphoreType.DMA((NBUF,)),
            pltpu.SemaphoreType.DMA((NBUF,)),
        ],
        compiler_params=pltpu.CompilerParams(use_tc_tiling_on_sc=False),
    )
    def k(w_hbm, idx_hbm, out_hbm, idx_v, rows_v, gsem, ssem):
        wid = lax.axis_index("s") * nc + lax.axis_index("c")
        base = wid * (ngroups * GROUP)
        pltpu.sync_copy(idx_hbm.at[wid], idx_v)

        def gather_start(g, b):
            pltpu.async_copy(w_hbm.at[idx_v.at[g]], rows_v.at[b], gsem.at[b])

        def gather_wait(g, b):
            pltpu.make_async_copy(w_hbm.at[idx_v.at[g]], rows_v.at[b],
                                  gsem.at[b]).wait()

        def scat_start(g, b):
            pltpu.async_copy(rows_v.at[b],
                             out_hbm.at[pl.ds(base + g * GROUP, GROUP)],
                             ssem.at[b])

        def scat_wait(g, b):
            pltpu.make_async_copy(rows_v.at[b],
                                  out_hbm.at[pl.ds(base + g * GROUP, GROUP)],
                                  ssem.at[b]).wait()

        for b in range(K):
            gather_start(b, b)

        def outer(t, _):
            for j in range(NBUF):
                g = t * NBUF + j
                gather_wait(g, j)
                scat_start(g, j)
                gn = g + K
                bn = (j + K) % NBUF

                @pl.when(gn < ngroups)
                def _():
                    @pl.when(gn >= NBUF)
                    def _():
                        scat_wait(gn - NBUF, bn)

                    gather_start(gn, bn)

            return 0

        lax.fori_loop(0, ngroups // NBUF, outer, 0)
        for b in range(NBUF):
            scat_wait(ngroups - NBUF + b, b)

    return k(w_pad, idx_grouped)


def _tc_pad_table_body(wt_ref, o_ref):
    blk = wt_ref[...]
    o_ref[...] = jnp.concatenate(
        [blk.T, jnp.zeros((blk.shape[1], PADW - DM), jnp.float32)], axis=1)


@jax.jit
def _tc_pad_table(wt):
    v = wt.shape[1]
    grid = (v + TBLK - 1) // TBLK
    return pl.pallas_call(
        _tc_pad_table_body,
        grid=(grid,),
        in_specs=[pl.BlockSpec((DM, TBLK), lambda c: (0, c))],
        out_specs=pl.BlockSpec((TBLK, PADW), lambda c: (c, 0)),
        out_shape=jax.ShapeDtypeStruct((v, PADW), jnp.float32),
    )(wt)


def _tc_transpose_body(x_ref, o_ref):
    blk = x_ref[0]
    xt = blk[:, :DM].T
    o_ref[0] = xt.reshape(8, 8, GROUP, GROUP).transpose(0, 2, 1, 3)


@jax.jit
def _tc_transpose(x):
    bsz = x.shape[1]
    nbj = bsz // GROUP
    return pl.pallas_call(
        _tc_transpose_body,
        grid=(HIST,),
        in_specs=[pl.BlockSpec((1, bsz, PADW), lambda h: (h, 0, 0))],
        out_specs=pl.BlockSpec((1, 8, nbj, 8, GROUP),
                               lambda h: (h, 0, 0, 0, 0)),
        out_shape=jax.ShapeDtypeStruct((HIST, 8, nbj, 8, GROUP), jnp.float32),
    )(x)


def kernel(token_ids, weight):
    bsz, h = token_ids.shape
    total = bsz * h
    ngroups = total // (NW * GROUP)
    idx = token_ids.astype(jnp.int32).T.reshape(NW, ngroups, GROUP)
    w_pad = _tc_pad_table(weight.T)
    rows = _sc_gather(w_pad, idx, ngroups)
    out5 = _tc_transpose(rows.reshape(h, bsz, PADW))
    return out5.transpose(2, 4, 0, 1, 3).reshape(bsz, h, DM)
